# Initial kernel scaffold; baseline (speedup 1.0000x reference)
import jax
import jax.numpy as jnp
from jax.experimental import pallas as pl


def kernel(x, edge_index, W1l, b1l, W1r, W2l, b2l, W2r, Wc, bc):
    raise NotImplementedError("write your pallas kernel here")


# speedup vs baseline: 3.7388x; 3.7388x over previous
import jax
import jax.numpy as jnp
from jax import lax
from jax.experimental import pallas as pl
from jax.experimental.pallas import tpu as pltpu
from jax.experimental.pallas import tpu_sc as plsc

N_NODES = 10000
N_EDGES = 320000
D_IN = 128
D_HID = 256

NC = 2
NS = 16
NW = NC * NS

NPAD = 10240
RPT = NPAD // NS
CH = 64
NB = 2
NBC = 8
EPW = 10240
EPAD = EPW * NW
ROWS_PW = EPW // CH
NGRP = ROWS_PW // NB
NGRPC = ROWS_PW // NBC

_F32 = jnp.float32

_MESH = plsc.VectorSubcoreMesh(core_axis_name="c", subcore_axis_name="s",
                               num_cores=NC, num_subcores=NS)


def _agg():
    def body(x_hbm, src_hbm, dst_hbm, zrow_hbm, sums_out, acc_sh, srcb,
             dstb, rows_v, sem):
        cid = lax.axis_index("c")
        sid = lax.axis_index("s")
        wid = sid * NC + cid
        pltpu.sync_copy(zrow_hbm, acc_sh.at[pl.ds(sid * RPT, RPT)])
        plsc.subcore_barrier()
        base_row = wid * ROWS_PW

        @pl.loop(0, NGRP)
        def _(g):
            r0 = base_row + g * NB
            pltpu.sync_copy(src_hbm.at[pl.ds(r0, NB)], srcb)
            pltpu.sync_copy(dst_hbm.at[pl.ds(r0, NB)], dstb)
            for b in range(NB):
                pltpu.async_copy(x_hbm.at[srcb.at[b]], rows_v.at[b],
                                 sem).wait()
                pltpu.sync_copy(rows_v.at[b], acc_sh.at[dstb.at[b]],
                                add=True)

        plsc.subcore_barrier()
        pltpu.sync_copy(acc_sh.at[pl.ds(sid * RPT, RPT)],
                        sums_out.at[cid, pl.ds(sid * RPT, RPT)])

    return pl.kernel(
        body,
        out_type=jax.ShapeDtypeStruct((NC, NPAD, D_IN), _F32),
        mesh=_MESH,
        scratch_types=(
            pltpu.VMEM_SHARED((NPAD, D_IN), _F32),
            pltpu.VMEM((NB, CH), jnp.int32),
            pltpu.VMEM((NB, CH), jnp.int32),
            pltpu.VMEM((NB, CH, D_IN), _F32),
            pltpu.SemaphoreType.DMA,
        ),
    )


def _count():
    def body(dst_hbm, ones_hbm, zrow_hbm, cnts_out, acc_sh, dstb, ones_v,
             sem):
        cid = lax.axis_index("c")
        sid = lax.axis_index("s")
        wid = sid * NC + cid
        pltpu.sync_copy(zrow_hbm, acc_sh.at[pl.ds(sid * RPT, RPT)])
        pltpu.sync_copy(ones_hbm, ones_v)
        plsc.subcore_barrier()
        base_row = wid * ROWS_PW

        @pl.loop(0, NGRPC)
        def _(g):
            r0 = base_row + g * NBC
            pltpu.sync_copy(dst_hbm.at[pl.ds(r0, NBC)], dstb)
            for b in range(NBC):
                pltpu.sync_copy(ones_v, acc_sh.at[dstb.at[b]], add=True)

        plsc.subcore_barrier()
        pltpu.sync_copy(acc_sh.at[pl.ds(sid * RPT, RPT)],
                        cnts_out.at[cid, pl.ds(sid * RPT, RPT)])

    return pl.kernel(
        body,
        out_type=jax.ShapeDtypeStruct((NC, NPAD, D_IN), _F32),
        mesh=_MESH,
        scratch_types=(
            pltpu.VMEM_SHARED((NPAD, D_IN), _F32),
            pltpu.VMEM((NBC, CH), jnp.int32),
            pltpu.VMEM((CH, D_IN), _F32),
            pltpu.SemaphoreType.DMA,
        ),
    )


RB = 1024


def _tc1_body(sums, cnts, x, w1l, b1l, w1r, h0, h1):
    s = sums[0] + sums[1]
    c = jnp.maximum(cnts[0][:, 0:1] + cnts[1][:, 0:1], 1.0)
    mean = s / c
    t = (jnp.dot(mean, w1l[...], preferred_element_type=_F32) + b1l[...]
         + jnp.dot(x[...], w1r[...], preferred_element_type=_F32))
    h = jnp.maximum(t, 0.0)
    h0[...] = h[:, :D_IN]
    h1[...] = h[:, D_IN:]


def _tc1(sums, cnts, x, w1l, b1l, w1r):
    return pl.pallas_call(
        _tc1_body,
        grid=(NPAD // RB,),
        in_specs=[
            pl.BlockSpec((NC, RB, D_IN), lambda i: (0, i, 0)),
            pl.BlockSpec((NC, RB, D_IN), lambda i: (0, i, 0)),
            pl.BlockSpec((RB, D_IN), lambda i: (i, 0)),
            pl.BlockSpec((D_IN, D_HID), lambda i: (0, 0)),
            pl.BlockSpec((1, D_HID), lambda i: (0, 0)),
            pl.BlockSpec((D_IN, D_HID), lambda i: (0, 0)),
        ],
        out_specs=[
            pl.BlockSpec((RB, D_IN), lambda i: (i, 0)),
            pl.BlockSpec((RB, D_IN), lambda i: (i, 0)),
        ],
        out_shape=[
            jax.ShapeDtypeStruct((NPAD, D_IN), _F32),
            jax.ShapeDtypeStruct((NPAD, D_IN), _F32),
        ],
    )(sums, cnts, x, w1l, b1l, w1r)


def _tc2_body(sa, sb, cnts, h0, h1, w2l, b2l, w2r, wc, bc, out):
    c = jnp.maximum(cnts[0][:, 0:1] + cnts[1][:, 0:1], 1.0)
    ma = (sa[0] + sa[1]) / c
    mb = (sb[0] + sb[1]) / c
    w2l_ = w2l[...]
    w2r_ = w2r[...]
    t = (jnp.dot(ma, w2l_[:D_IN], preferred_element_type=_F32)
         + jnp.dot(mb, w2l_[D_IN:], preferred_element_type=_F32)
         + b2l[...]
         + jnp.dot(h0[...], w2r_[:D_IN], preferred_element_type=_F32)
         + jnp.dot(h1[...], w2r_[D_IN:], preferred_element_type=_F32))
    h2 = jnp.maximum(t, 0.0)
    out[...] = jnp.dot(h2, wc[...], preferred_element_type=_F32) + bc[...]


def _tc2(sa, sb, cnts, h0, h1, w2l, b2l, w2r, wc, bc):
    return pl.pallas_call(
        _tc2_body,
        grid=(NPAD // RB,),
        in_specs=[
            pl.BlockSpec((NC, RB, D_IN), lambda i: (0, i, 0)),
            pl.BlockSpec((NC, RB, D_IN), lambda i: (0, i, 0)),
            pl.BlockSpec((NC, RB, D_IN), lambda i: (0, i, 0)),
            pl.BlockSpec((RB, D_IN), lambda i: (i, 0)),
            pl.BlockSpec((RB, D_IN), lambda i: (i, 0)),
            pl.BlockSpec((D_HID, D_HID), lambda i: (0, 0)),
            pl.BlockSpec((1, D_HID), lambda i: (0, 0)),
            pl.BlockSpec((D_HID, D_HID), lambda i: (0, 0)),
            pl.BlockSpec((D_HID, D_IN), lambda i: (0, 0)),
            pl.BlockSpec((1, D_IN), lambda i: (0, 0)),
        ],
        out_specs=pl.BlockSpec((RB, D_IN), lambda i: (i, 0)),
        out_shape=jax.ShapeDtypeStruct((NPAD, D_IN), _F32),
    )(sa, sb, cnts, h0, h1, w2l, b2l, w2r, wc, bc)


def kernel(x, edge_index, W1l, b1l, W1r, W2l, b2l, W2r, Wc, bc):
    src = edge_index[0].astype(jnp.int32)
    dst = edge_index[1].astype(jnp.int32)

    n_extra = EPAD - N_EDGES
    ar = jnp.arange(n_extra, dtype=jnp.int32)
    src_p = jnp.concatenate([src, ar % N_NODES])
    dst_p = jnp.concatenate([dst, N_NODES + ar % (NPAD - N_NODES)])
    src2d = src_p.reshape(EPAD // CH, CH)
    dst2d = dst_p.reshape(EPAD // CH, CH)

    x_p = jnp.zeros((NPAD, D_IN), _F32).at[:N_NODES].set(x)
    zrow = jnp.zeros((RPT, D_IN), _F32)
    ones = jnp.ones((CH, D_IN), _F32)

    agg = _agg()
    cnts = _count()(dst2d, ones, zrow)
    sums1 = agg(x_p, src2d, dst2d, zrow)

    b1l2 = b1l.reshape(1, D_HID)
    b2l2 = b2l.reshape(1, D_HID)
    h0, h1 = _tc1(sums1, cnts, x_p, W1l, b1l2, W1r)

    s2a = agg(h0, src2d, dst2d, zrow)
    s2b = agg(h1, src2d, dst2d, zrow)

    wc_p = jnp.zeros((D_HID, D_IN), _F32).at[:, :1].set(Wc)
    bc_p = jnp.zeros((1, D_IN), _F32).at[:, :1].set(bc.reshape(1, 1))
    out = _tc2(s2a, s2b, cnts, h0, h1, W2l, b2l2, W2r, wc_p, bc_p)
    return out[:N_NODES, :1]


# speedup vs baseline: 5.6532x; 1.5120x over previous
import jax
import jax.numpy as jnp
from jax import lax
from jax.experimental import pallas as pl
from jax.experimental.pallas import tpu as pltpu
from jax.experimental.pallas import tpu_sc as plsc

N_NODES = 10000
N_EDGES = 320000
D_IN = 128
D_HID = 256

NC = 2
NS = 16
NW = NC * NS

NPAD = 10240
RPT = NPAD // NS
CH = 64
NBI = 16
NBC = 8
EPW = 10240
EPAD = EPW * NW
ROWS_PW = EPW // CH
NGO = ROWS_PW // NBI
NGRPC = ROWS_PW // NBC

_F32 = jnp.float32

_MESH = plsc.VectorSubcoreMesh(core_axis_name="c", subcore_axis_name="s",
                               num_cores=NC, num_subcores=NS)


def _zero_stripe(buf, acc_sh, sid):
    z16 = jnp.zeros((16,), _F32)

    @pl.loop(0, CH)
    def _(i):
        for j in range(D_IN // 16):
            buf[i, pl.ds(j * 16, 16)] = z16

    for r in range(RPT // CH):
        pltpu.sync_copy(buf, acc_sh.at[pl.ds(sid * RPT + r * CH, CH)])


def _agg():
    def body(x_hbm, src_hbm, dst_hbm, sums_out, acc_sh, srcb, dstb, rows_v,
             gsem0, gsem1, ssem0, ssem1):
        cid = lax.axis_index("c")
        sid = lax.axis_index("s")
        wid = sid * NC + cid
        _zero_stripe(rows_v.at[0], acc_sh, sid)
        plsc.subcore_barrier()
        base_row = wid * ROWS_PW
        gsems = (gsem0, gsem1)
        ssems = (ssem0, ssem1)

        @pl.loop(0, NGO)
        def _(go):
            r0 = base_row + go * NBI
            pltpu.sync_copy(src_hbm.at[pl.ds(r0, NBI)], srcb)
            pltpu.sync_copy(dst_hbm.at[pl.ds(r0, NBI)], dstb)
            gath = [None, None]
            scat = [None, None]
            gath[0] = pltpu.async_copy(x_hbm.at[srcb.at[0]], rows_v.at[0],
                                       gsem0)
            for b in range(NBI):
                p = b & 1
                q = p ^ 1
                gath[p].wait()
                scat[p] = pltpu.async_copy(rows_v.at[p],
                                           acc_sh.at[dstb.at[b]],
                                           ssems[p], add=True)
                if b + 1 < NBI:
                    if scat[q] is not None:
                        scat[q].wait()
                    gath[q] = pltpu.async_copy(x_hbm.at[srcb.at[b + 1]],
                                               rows_v.at[q], gsems[q])
            scat[0].wait()
            scat[1].wait()

        plsc.subcore_barrier()
        pltpu.sync_copy(acc_sh.at[pl.ds(sid * RPT, RPT)],
                        sums_out.at[cid, pl.ds(sid * RPT, RPT)])

    return pl.kernel(
        body,
        out_type=jax.ShapeDtypeStruct((NC, NPAD, D_IN), _F32),
        mesh=_MESH,
        scratch_types=(
            pltpu.VMEM_SHARED((NPAD, D_IN), _F32),
            pltpu.VMEM((NBI, CH), jnp.int32),
            pltpu.VMEM((NBI, CH), jnp.int32),
            pltpu.VMEM((2, CH, D_IN), _F32),
            pltpu.SemaphoreType.DMA,
            pltpu.SemaphoreType.DMA,
            pltpu.SemaphoreType.DMA,
            pltpu.SemaphoreType.DMA,
        ),
    )


def _count():
    def body(dst_hbm, cnts_out, acc_sh, dstb, ones_v, sem):
        cid = lax.axis_index("c")
        sid = lax.axis_index("s")
        wid = sid * NC + cid
        _zero_stripe(ones_v, acc_sh, sid)
        one16 = jnp.ones((16,), _F32)

        @pl.loop(0, CH)
        def _(i):
            for j in range(D_IN // 16):
                ones_v[i, pl.ds(j * 16, 16)] = one16

        plsc.subcore_barrier()
        base_row = wid * ROWS_PW

        @pl.loop(0, NGRPC)
        def _(g):
            r0 = base_row + g * NBC
            pltpu.sync_copy(dst_hbm.at[pl.ds(r0, NBC)], dstb)
            descs = [pltpu.async_copy(ones_v, acc_sh.at[dstb.at[b]], sem,
                                      add=True) for b in range(NBC)]
            for d in descs:
                d.wait()

        plsc.subcore_barrier()
        pltpu.sync_copy(acc_sh.at[pl.ds(sid * RPT, RPT)],
                        cnts_out.at[cid, pl.ds(sid * RPT, RPT)])

    return pl.kernel(
        body,
        out_type=jax.ShapeDtypeStruct((NC, NPAD, D_IN), _F32),
        mesh=_MESH,
        scratch_types=(
            pltpu.VMEM_SHARED((NPAD, D_IN), _F32),
            pltpu.VMEM((NBC, CH), jnp.int32),
            pltpu.VMEM((CH, D_IN), _F32),
            pltpu.SemaphoreType.DMA,
        ),
    )


RB = 1024


def _tc1_body(sums, cnts, x, w1l, b1l, w1r, h0, h1):
    s = sums[0] + sums[1]
    c = jnp.maximum(cnts[0][:, 0:1] + cnts[1][:, 0:1], 1.0)
    mean = s / c
    t = (jnp.dot(mean, w1l[...], preferred_element_type=_F32) + b1l[...]
         + jnp.dot(x[...], w1r[...], preferred_element_type=_F32))
    h = jnp.maximum(t, 0.0)
    h0[...] = h[:, :D_IN]
    h1[...] = h[:, D_IN:]


def _tc1(sums, cnts, x, w1l, b1l, w1r):
    return pl.pallas_call(
        _tc1_body,
        grid=(NPAD // RB,),
        in_specs=[
            pl.BlockSpec((NC, RB, D_IN), lambda i: (0, i, 0)),
            pl.BlockSpec((NC, RB, D_IN), lambda i: (0, i, 0)),
            pl.BlockSpec((RB, D_IN), lambda i: (i, 0)),
            pl.BlockSpec((D_IN, D_HID), lambda i: (0, 0)),
            pl.BlockSpec((1, D_HID), lambda i: (0, 0)),
            pl.BlockSpec((D_IN, D_HID), lambda i: (0, 0)),
        ],
        out_specs=[
            pl.BlockSpec((RB, D_IN), lambda i: (i, 0)),
            pl.BlockSpec((RB, D_IN), lambda i: (i, 0)),
        ],
        out_shape=[
            jax.ShapeDtypeStruct((NPAD, D_IN), _F32),
            jax.ShapeDtypeStruct((NPAD, D_IN), _F32),
        ],
    )(sums, cnts, x, w1l, b1l, w1r)


def _tc2_body(sa, sb, cnts, h0, h1, w2l, b2l, w2r, wc, bc, out):
    c = jnp.maximum(cnts[0][:, 0:1] + cnts[1][:, 0:1], 1.0)
    ma = (sa[0] + sa[1]) / c
    mb = (sb[0] + sb[1]) / c
    w2l_ = w2l[...]
    w2r_ = w2r[...]
    t = (jnp.dot(ma, w2l_[:D_IN], preferred_element_type=_F32)
         + jnp.dot(mb, w2l_[D_IN:], preferred_element_type=_F32)
         + b2l[...]
         + jnp.dot(h0[...], w2r_[:D_IN], preferred_element_type=_F32)
         + jnp.dot(h1[...], w2r_[D_IN:], preferred_element_type=_F32))
    h2 = jnp.maximum(t, 0.0)
    out[...] = jnp.dot(h2, wc[...], preferred_element_type=_F32) + bc[...]


def _tc2(sa, sb, cnts, h0, h1, w2l, b2l, w2r, wc, bc):
    return pl.pallas_call(
        _tc2_body,
        grid=(NPAD // RB,),
        in_specs=[
            pl.BlockSpec((NC, RB, D_IN), lambda i: (0, i, 0)),
            pl.BlockSpec((NC, RB, D_IN), lambda i: (0, i, 0)),
            pl.BlockSpec((NC, RB, D_IN), lambda i: (0, i, 0)),
            pl.BlockSpec((RB, D_IN), lambda i: (i, 0)),
            pl.BlockSpec((RB, D_IN), lambda i: (i, 0)),
            pl.BlockSpec((D_HID, D_HID), lambda i: (0, 0)),
            pl.BlockSpec((1, D_HID), lambda i: (0, 0)),
            pl.BlockSpec((D_HID, D_HID), lambda i: (0, 0)),
            pl.BlockSpec((D_HID, D_IN), lambda i: (0, 0)),
            pl.BlockSpec((1, D_IN), lambda i: (0, 0)),
        ],
        out_specs=pl.BlockSpec((RB, D_IN), lambda i: (i, 0)),
        out_shape=jax.ShapeDtypeStruct((NPAD, D_IN), _F32),
    )(sa, sb, cnts, h0, h1, w2l, b2l, w2r, wc, bc)


def kernel(x, edge_index, W1l, b1l, W1r, W2l, b2l, W2r, Wc, bc):
    src = edge_index[0].astype(jnp.int32)
    dst = edge_index[1].astype(jnp.int32)

    n_extra = EPAD - N_EDGES
    ar = jnp.arange(n_extra, dtype=jnp.int32)
    src_p = jnp.concatenate([src, ar % N_NODES])
    dst_p = jnp.concatenate([dst, N_NODES + ar % (NPAD - N_NODES)])
    src2d = src_p.reshape(EPAD // CH, CH)
    dst2d = dst_p.reshape(EPAD // CH, CH)

    x_p = jnp.zeros((NPAD, D_IN), _F32).at[:N_NODES].set(x)

    agg = _agg()
    cnts = _count()(dst2d)
    sums1 = agg(x_p, src2d, dst2d)

    b1l2 = b1l.reshape(1, D_HID)
    b2l2 = b2l.reshape(1, D_HID)
    h0, h1 = _tc1(sums1, cnts, x_p, W1l, b1l2, W1r)

    s2a = agg(h0, src2d, dst2d)
    s2b = agg(h1, src2d, dst2d)

    wc_p = jnp.zeros((D_HID, D_IN), _F32).at[:, :1].set(Wc)
    bc_p = jnp.zeros((1, D_IN), _F32).at[:, :1].set(bc.reshape(1, 1))
    out = _tc2(s2a, s2b, cnts, h0, h1, W2l, b2l2, W2r, wc_p, bc_p)
    return out[:N_NODES, :1]


# speedup vs baseline: 7.7512x; 1.3711x over previous
import jax
import jax.numpy as jnp
from jax import lax
from jax.experimental import pallas as pl
from jax.experimental.pallas import tpu as pltpu
from jax.experimental.pallas import tpu_sc as plsc

N_NODES = 10000
N_EDGES = 320000
D_IN = 128
D_HID = 256

NC = 2
NS = 16
NW = NC * NS

NPAD = 10240
RPT = NPAD // NS
CH = 64
NBUF = 4
NBI = 16
NBC = 8
EPW = 10240
EPAD = EPW * NW
ROWS_PW = EPW // CH
NGO = ROWS_PW // NBI
NGRPC = ROWS_PW // NBC

_F32 = jnp.float32

_MESH = plsc.VectorSubcoreMesh(core_axis_name="c", subcore_axis_name="s",
                               num_cores=NC, num_subcores=NS)


def _zero_stripe(buf, acc_sh, sid):
    z16 = jnp.zeros((16,), _F32)

    @pl.loop(0, CH)
    def _(i):
        for j in range(D_IN // 16):
            buf[i, pl.ds(j * 16, 16)] = z16

    for r in range(RPT // CH):
        pltpu.sync_copy(buf, acc_sh.at[pl.ds(sid * RPT + r * CH, CH)])


def _agg():
    def body(x_hbm, src_hbm, dst_hbm, sums_out, acc_sh, srcb, dstb, rows_v,
             gsem0, gsem1, gsem2, gsem3, ssem0, ssem1, ssem2, ssem3):
        cid = lax.axis_index("c")
        sid = lax.axis_index("s")
        wid = sid * NC + cid
        _zero_stripe(rows_v.at[0], acc_sh, sid)
        plsc.subcore_barrier()
        base_row = wid * ROWS_PW
        gsems = (gsem0, gsem1, gsem2, gsem3)
        ssems = (ssem0, ssem1, ssem2, ssem3)

        @pl.loop(0, NGO)
        def _(go):
            r0 = base_row + go * NBI
            pltpu.sync_copy(src_hbm.at[pl.ds(r0, NBI)], srcb)
            pltpu.sync_copy(dst_hbm.at[pl.ds(r0, NBI)], dstb)
            gath = [None] * NBUF
            scat = [None] * NBUF
            for j in range(NBUF - 1):
                gath[j] = pltpu.async_copy(x_hbm.at[srcb.at[j]],
                                           rows_v.at[j], gsems[j])
            for b in range(NBI):
                p = b % NBUF
                gath[p].wait()
                scat[p] = pltpu.async_copy(rows_v.at[p],
                                           acc_sh.at[dstb.at[b]],
                                           ssems[p], add=True)
                nb = b + NBUF - 1
                if nb < NBI:
                    np_ = nb % NBUF
                    if scat[np_] is not None:
                        scat[np_].wait()
                    gath[np_] = pltpu.async_copy(x_hbm.at[srcb.at[nb]],
                                                 rows_v.at[np_], gsems[np_])
            for b in range(NBI - NBUF, NBI):
                scat[b % NBUF].wait()

        plsc.subcore_barrier()
        pltpu.sync_copy(acc_sh.at[pl.ds(sid * RPT, RPT)],
                        sums_out.at[cid, pl.ds(sid * RPT, RPT)])

    return pl.kernel(
        body,
        out_type=jax.ShapeDtypeStruct((NC, NPAD, D_IN), _F32),
        mesh=_MESH,
        scratch_types=(
            pltpu.VMEM_SHARED((NPAD, D_IN), _F32),
            pltpu.VMEM((NBI, CH), jnp.int32),
            pltpu.VMEM((NBI, CH), jnp.int32),
            pltpu.VMEM((NBUF, CH, D_IN), _F32),
            pltpu.SemaphoreType.DMA,
            pltpu.SemaphoreType.DMA,
            pltpu.SemaphoreType.DMA,
            pltpu.SemaphoreType.DMA,
            pltpu.SemaphoreType.DMA,
            pltpu.SemaphoreType.DMA,
            pltpu.SemaphoreType.DMA,
            pltpu.SemaphoreType.DMA,
        ),
    )


def _count():
    def body(dst_hbm, cnts_out, acc_sh, dstb, ones_v, sem):
        cid = lax.axis_index("c")
        sid = lax.axis_index("s")
        wid = sid * NC + cid
        _zero_stripe(ones_v, acc_sh, sid)
        one16 = jnp.ones((16,), _F32)

        @pl.loop(0, CH)
        def _(i):
            for j in range(D_IN // 16):
                ones_v[i, pl.ds(j * 16, 16)] = one16

        plsc.subcore_barrier()
        base_row = wid * ROWS_PW

        @pl.loop(0, NGRPC)
        def _(g):
            r0 = base_row + g * NBC
            pltpu.sync_copy(dst_hbm.at[pl.ds(r0, NBC)], dstb)
            descs = [pltpu.async_copy(ones_v, acc_sh.at[dstb.at[b]], sem,
                                      add=True) for b in range(NBC)]
            for d in descs:
                d.wait()

        plsc.subcore_barrier()
        pltpu.sync_copy(acc_sh.at[pl.ds(sid * RPT, RPT)],
                        cnts_out.at[cid, pl.ds(sid * RPT, RPT)])

    return pl.kernel(
        body,
        out_type=jax.ShapeDtypeStruct((NC, NPAD, D_IN), _F32),
        mesh=_MESH,
        scratch_types=(
            pltpu.VMEM_SHARED((NPAD, D_IN), _F32),
            pltpu.VMEM((NBC, CH), jnp.int32),
            pltpu.VMEM((CH, D_IN), _F32),
            pltpu.SemaphoreType.DMA,
        ),
    )


RB = 1024


def _tc1_body(sums, cnts, x, w1l, b1l, w1r, h0, h1):
    s = sums[0] + sums[1]
    c = jnp.maximum(cnts[0][:, 0:1] + cnts[1][:, 0:1], 1.0)
    mean = s / c
    t = (jnp.dot(mean, w1l[...], preferred_element_type=_F32) + b1l[...]
         + jnp.dot(x[...], w1r[...], preferred_element_type=_F32))
    h = jnp.maximum(t, 0.0)
    h0[...] = h[:, :D_IN]
    h1[...] = h[:, D_IN:]


def _tc1(sums, cnts, x, w1l, b1l, w1r):
    return pl.pallas_call(
        _tc1_body,
        grid=(NPAD // RB,),
        in_specs=[
            pl.BlockSpec((NC, RB, D_IN), lambda i: (0, i, 0)),
            pl.BlockSpec((NC, RB, D_IN), lambda i: (0, i, 0)),
            pl.BlockSpec((RB, D_IN), lambda i: (i, 0)),
            pl.BlockSpec((D_IN, D_HID), lambda i: (0, 0)),
            pl.BlockSpec((1, D_HID), lambda i: (0, 0)),
            pl.BlockSpec((D_IN, D_HID), lambda i: (0, 0)),
        ],
        out_specs=[
            pl.BlockSpec((RB, D_IN), lambda i: (i, 0)),
            pl.BlockSpec((RB, D_IN), lambda i: (i, 0)),
        ],
        out_shape=[
            jax.ShapeDtypeStruct((NPAD, D_IN), _F32),
            jax.ShapeDtypeStruct((NPAD, D_IN), _F32),
        ],
    )(sums, cnts, x, w1l, b1l, w1r)


def _tc2_body(sa, sb, cnts, h0, h1, w2l, b2l, w2r, wc, bc, out):
    c = jnp.maximum(cnts[0][:, 0:1] + cnts[1][:, 0:1], 1.0)
    ma = (sa[0] + sa[1]) / c
    mb = (sb[0] + sb[1]) / c
    w2l_ = w2l[...]
    w2r_ = w2r[...]
    t = (jnp.dot(ma, w2l_[:D_IN], preferred_element_type=_F32)
         + jnp.dot(mb, w2l_[D_IN:], preferred_element_type=_F32)
         + b2l[...]
         + jnp.dot(h0[...], w2r_[:D_IN], preferred_element_type=_F32)
         + jnp.dot(h1[...], w2r_[D_IN:], preferred_element_type=_F32))
    h2 = jnp.maximum(t, 0.0)
    out[...] = jnp.dot(h2, wc[...], preferred_element_type=_F32) + bc[...]


def _tc2(sa, sb, cnts, h0, h1, w2l, b2l, w2r, wc, bc):
    return pl.pallas_call(
        _tc2_body,
        grid=(NPAD // RB,),
        in_specs=[
            pl.BlockSpec((NC, RB, D_IN), lambda i: (0, i, 0)),
            pl.BlockSpec((NC, RB, D_IN), lambda i: (0, i, 0)),
            pl.BlockSpec((NC, RB, D_IN), lambda i: (0, i, 0)),
            pl.BlockSpec((RB, D_IN), lambda i: (i, 0)),
            pl.BlockSpec((RB, D_IN), lambda i: (i, 0)),
            pl.BlockSpec((D_HID, D_HID), lambda i: (0, 0)),
            pl.BlockSpec((1, D_HID), lambda i: (0, 0)),
            pl.BlockSpec((D_HID, D_HID), lambda i: (0, 0)),
            pl.BlockSpec((D_HID, D_IN), lambda i: (0, 0)),
            pl.BlockSpec((1, D_IN), lambda i: (0, 0)),
        ],
        out_specs=pl.BlockSpec((RB, D_IN), lambda i: (i, 0)),
        out_shape=jax.ShapeDtypeStruct((NPAD, D_IN), _F32),
    )(sa, sb, cnts, h0, h1, w2l, b2l, w2r, wc, bc)


def kernel(x, edge_index, W1l, b1l, W1r, W2l, b2l, W2r, Wc, bc):
    src = edge_index[0].astype(jnp.int32)
    dst = edge_index[1].astype(jnp.int32)

    n_extra = EPAD - N_EDGES
    ar = jnp.arange(n_extra, dtype=jnp.int32)
    src_p = jnp.concatenate([src, ar % N_NODES])
    dst_p = jnp.concatenate([dst, N_NODES + ar % (NPAD - N_NODES)])
    src2d = src_p.reshape(EPAD // CH, CH)
    dst2d = dst_p.reshape(EPAD // CH, CH)

    x_p = jnp.zeros((NPAD, D_IN), _F32).at[:N_NODES].set(x)

    agg = _agg()
    cnts = _count()(dst2d)
    sums1 = agg(x_p, src2d, dst2d)

    b1l2 = b1l.reshape(1, D_HID)
    b2l2 = b2l.reshape(1, D_HID)
    h0, h1 = _tc1(sums1, cnts, x_p, W1l, b1l2, W1r)

    s2a = agg(h0, src2d, dst2d)
    s2b = agg(h1, src2d, dst2d)

    wc_p = jnp.zeros((D_HID, D_IN), _F32).at[:, :1].set(Wc)
    bc_p = jnp.zeros((1, D_IN), _F32).at[:, :1].set(bc.reshape(1, 1))
    out = _tc2(s2a, s2b, cnts, h0, h1, W2l, b2l2, W2r, wc_p, bc_p)
    return out[:N_NODES, :1]


# speedup vs baseline: 8.0616x; 1.0400x over previous
import jax
import jax.numpy as jnp
from jax import lax
from jax.experimental import pallas as pl
from jax.experimental.pallas import tpu as pltpu
from jax.experimental.pallas import tpu_sc as plsc

N_NODES = 10000
N_EDGES = 320000
D_IN = 128
D_HID = 256

NC = 2
NS = 16
NW = NC * NS

NPAD = 10240
RPT = NPAD // NS
CH = 64
NBUF = 4
NBI = 16
NBC = 8
EPW = 10240
EPAD = EPW * NW
ROWS_PW = EPW // CH
NGO = ROWS_PW // NBI
NGRPC = ROWS_PW // NBC

_F32 = jnp.float32

_MESH = plsc.VectorSubcoreMesh(core_axis_name="c", subcore_axis_name="s",
                               num_cores=NC, num_subcores=NS)


def _zero_stripe(buf, acc_sh, sid):
    z16 = jnp.zeros((16,), _F32)

    @pl.loop(0, CH)
    def _(i):
        for j in range(D_IN // 16):
            buf[i, pl.ds(j * 16, 16)] = z16

    for r in range(RPT // CH):
        pltpu.sync_copy(buf, acc_sh.at[pl.ds(sid * RPT + r * CH, CH)])


def _agg():
    def body(x_hbm, src_hbm, dst_hbm, sums_out, acc_sh, srcb, dstb, rows_v,
             gsem0, gsem1, gsem2, gsem3, ssem0, ssem1, ssem2, ssem3):
        cid = lax.axis_index("c")
        sid = lax.axis_index("s")
        wid = sid * NC + cid
        _zero_stripe(rows_v.at[0], acc_sh, sid)
        plsc.subcore_barrier()
        base_row = wid * ROWS_PW
        gsems = (gsem0, gsem1, gsem2, gsem3)
        ssems = (ssem0, ssem1, ssem2, ssem3)

        @pl.loop(0, NGO)
        def _(go):
            r0 = base_row + go * NBI
            pltpu.sync_copy(src_hbm.at[pl.ds(r0, NBI)], srcb)
            pltpu.sync_copy(dst_hbm.at[pl.ds(r0, NBI)], dstb)
            gath = [None] * NBUF
            scat = [None] * NBUF
            for j in range(NBUF - 1):
                gath[j] = pltpu.async_copy(x_hbm.at[srcb.at[j]],
                                           rows_v.at[j], gsems[j])
            for b in range(NBI):
                p = b % NBUF
                gath[p].wait()
                scat[p] = pltpu.async_copy(rows_v.at[p],
                                           acc_sh.at[dstb.at[b]],
                                           ssems[p], add=True)
                nb = b + NBUF - 1
                if nb < NBI:
                    np_ = nb % NBUF
                    if scat[np_] is not None:
                        scat[np_].wait()
                    gath[np_] = pltpu.async_copy(x_hbm.at[srcb.at[nb]],
                                                 rows_v.at[np_], gsems[np_])
            for b in range(NBI - NBUF, NBI):
                scat[b % NBUF].wait()

        plsc.subcore_barrier()
        pltpu.sync_copy(acc_sh.at[pl.ds(sid * RPT, RPT)],
                        sums_out.at[cid, pl.ds(sid * RPT, RPT)])

    return pl.kernel(
        body,
        out_type=jax.ShapeDtypeStruct((NC, NPAD, D_IN), _F32),
        mesh=_MESH,
        scratch_types=(
            pltpu.VMEM_SHARED((NPAD, D_IN), _F32),
            pltpu.VMEM((NBI, CH), jnp.int32),
            pltpu.VMEM((NBI, CH), jnp.int32),
            pltpu.VMEM((NBUF, CH, D_IN), _F32),
            pltpu.SemaphoreType.DMA,
            pltpu.SemaphoreType.DMA,
            pltpu.SemaphoreType.DMA,
            pltpu.SemaphoreType.DMA,
            pltpu.SemaphoreType.DMA,
            pltpu.SemaphoreType.DMA,
            pltpu.SemaphoreType.DMA,
            pltpu.SemaphoreType.DMA,
        ),
    )


ROWS_PC = (EPAD // CH) // NS
NGO2 = ROWS_PC // NBI


def _agg2():
    def body(h_hbm, src_hbm, dst_hbm, sums_out, acc_sh, srcb, dstb, rows_v,
             gsem0, gsem1, gsem2, gsem3, ssem0, ssem1, ssem2, ssem3):
        cid = lax.axis_index("c")
        sid = lax.axis_index("s")
        _zero_stripe(rows_v.at[0], acc_sh, sid)
        plsc.subcore_barrier()
        base_row = sid * ROWS_PC
        gsems = (gsem0, gsem1, gsem2, gsem3)
        ssems = (ssem0, ssem1, ssem2, ssem3)

        @pl.loop(0, NGO2)
        def _(go):
            r0 = base_row + go * NBI
            pltpu.sync_copy(src_hbm.at[cid, pl.ds(r0, NBI)], srcb)
            pltpu.sync_copy(dst_hbm.at[pl.ds(r0, NBI)], dstb)
            gath = [None] * NBUF
            scat = [None] * NBUF
            for j in range(NBUF - 1):
                gath[j] = pltpu.async_copy(h_hbm.at[srcb.at[j]],
                                           rows_v.at[j], gsems[j])
            for b in range(NBI):
                p = b % NBUF
                gath[p].wait()
                scat[p] = pltpu.async_copy(rows_v.at[p],
                                           acc_sh.at[dstb.at[b]],
                                           ssems[p], add=True)
                nb = b + NBUF - 1
                if nb < NBI:
                    np_ = nb % NBUF
                    if scat[np_] is not None:
                        scat[np_].wait()
                    gath[np_] = pltpu.async_copy(h_hbm.at[srcb.at[nb]],
                                                 rows_v.at[np_], gsems[np_])
            for b in range(NBI - NBUF, NBI):
                scat[b % NBUF].wait()

        plsc.subcore_barrier()
        pltpu.sync_copy(acc_sh.at[pl.ds(sid * RPT, RPT)],
                        sums_out.at[cid, pl.ds(sid * RPT, RPT)])

    return pl.kernel(
        body,
        out_type=jax.ShapeDtypeStruct((NC, NPAD, D_IN), _F32),
        mesh=_MESH,
        scratch_types=(
            pltpu.VMEM_SHARED((NPAD, D_IN), _F32),
            pltpu.VMEM((NBI, CH), jnp.int32),
            pltpu.VMEM((NBI, CH), jnp.int32),
            pltpu.VMEM((NBUF, CH, D_IN), _F32),
            pltpu.SemaphoreType.DMA,
            pltpu.SemaphoreType.DMA,
            pltpu.SemaphoreType.DMA,
            pltpu.SemaphoreType.DMA,
            pltpu.SemaphoreType.DMA,
            pltpu.SemaphoreType.DMA,
            pltpu.SemaphoreType.DMA,
            pltpu.SemaphoreType.DMA,
        ),
    )


def _count():
    def body(dst_hbm, cnts_out, acc_sh, dstb, ones_v, sem):
        cid = lax.axis_index("c")
        sid = lax.axis_index("s")
        wid = sid * NC + cid
        _zero_stripe(ones_v, acc_sh, sid)
        one16 = jnp.ones((16,), _F32)

        @pl.loop(0, CH)
        def _(i):
            for j in range(D_IN // 16):
                ones_v[i, pl.ds(j * 16, 16)] = one16

        plsc.subcore_barrier()
        base_row = wid * ROWS_PW

        @pl.loop(0, NGRPC)
        def _(g):
            r0 = base_row + g * NBC
            pltpu.sync_copy(dst_hbm.at[pl.ds(r0, NBC)], dstb)
            descs = [pltpu.async_copy(ones_v, acc_sh.at[dstb.at[b]], sem,
                                      add=True) for b in range(NBC)]
            for d in descs:
                d.wait()

        plsc.subcore_barrier()
        pltpu.sync_copy(acc_sh.at[pl.ds(sid * RPT, RPT)],
                        cnts_out.at[cid, pl.ds(sid * RPT, RPT)])

    return pl.kernel(
        body,
        out_type=jax.ShapeDtypeStruct((NC, NPAD, D_IN), _F32),
        mesh=_MESH,
        scratch_types=(
            pltpu.VMEM_SHARED((NPAD, D_IN), _F32),
            pltpu.VMEM((NBC, CH), jnp.int32),
            pltpu.VMEM((CH, D_IN), _F32),
            pltpu.SemaphoreType.DMA,
        ),
    )


RB = 1024


def _tc1_body(sums, cnts, x, w1l, b1l, w1r, hout):
    s = sums[0] + sums[1]
    c = jnp.maximum(cnts[0][:, 0:1] + cnts[1][:, 0:1], 1.0)
    mean = s / c
    t = (jnp.dot(mean, w1l[...], preferred_element_type=_F32) + b1l[...]
         + jnp.dot(x[...], w1r[...], preferred_element_type=_F32))
    h = jnp.maximum(t, 0.0)
    hout[0] = h[:, :D_IN]
    hout[1] = h[:, D_IN:]


def _tc1(sums, cnts, x, w1l, b1l, w1r):
    return pl.pallas_call(
        _tc1_body,
        grid=(NPAD // RB,),
        in_specs=[
            pl.BlockSpec((NC, RB, D_IN), lambda i: (0, i, 0)),
            pl.BlockSpec((NC, RB, D_IN), lambda i: (0, i, 0)),
            pl.BlockSpec((RB, D_IN), lambda i: (i, 0)),
            pl.BlockSpec((D_IN, D_HID), lambda i: (0, 0)),
            pl.BlockSpec((1, D_HID), lambda i: (0, 0)),
            pl.BlockSpec((D_IN, D_HID), lambda i: (0, 0)),
        ],
        out_specs=pl.BlockSpec((2, RB, D_IN), lambda i: (0, i, 0)),
        out_shape=jax.ShapeDtypeStruct((2, NPAD, D_IN), _F32),
    )(sums, cnts, x, w1l, b1l, w1r)


def _tc2_body(s2, cnts, h, w2l, b2l, w2r, wc, bc, out):
    c = jnp.maximum(cnts[0][:, 0:1] + cnts[1][:, 0:1], 1.0)
    ma = s2[0] / c
    mb = s2[1] / c
    w2l_ = w2l[...]
    w2r_ = w2r[...]
    t = (jnp.dot(ma, w2l_[:D_IN], preferred_element_type=_F32)
         + jnp.dot(mb, w2l_[D_IN:], preferred_element_type=_F32)
         + b2l[...]
         + jnp.dot(h[0], w2r_[:D_IN], preferred_element_type=_F32)
         + jnp.dot(h[1], w2r_[D_IN:], preferred_element_type=_F32))
    h2 = jnp.maximum(t, 0.0)
    out[...] = jnp.dot(h2, wc[...], preferred_element_type=_F32) + bc[...]


def _tc2(s2, cnts, h, w2l, b2l, w2r, wc, bc):
    return pl.pallas_call(
        _tc2_body,
        grid=(NPAD // RB,),
        in_specs=[
            pl.BlockSpec((NC, RB, D_IN), lambda i: (0, i, 0)),
            pl.BlockSpec((NC, RB, D_IN), lambda i: (0, i, 0)),
            pl.BlockSpec((2, RB, D_IN), lambda i: (0, i, 0)),
            pl.BlockSpec((D_HID, D_HID), lambda i: (0, 0)),
            pl.BlockSpec((1, D_HID), lambda i: (0, 0)),
            pl.BlockSpec((D_HID, D_HID), lambda i: (0, 0)),
            pl.BlockSpec((D_HID, D_IN), lambda i: (0, 0)),
            pl.BlockSpec((1, D_IN), lambda i: (0, 0)),
        ],
        out_specs=pl.BlockSpec((RB, D_IN), lambda i: (i, 0)),
        out_shape=jax.ShapeDtypeStruct((NPAD, D_IN), _F32),
    )(s2, cnts, h, w2l, b2l, w2r, wc, bc)


def kernel(x, edge_index, W1l, b1l, W1r, W2l, b2l, W2r, Wc, bc):
    src = edge_index[0].astype(jnp.int32)
    dst = edge_index[1].astype(jnp.int32)

    n_extra = EPAD - N_EDGES
    ar = jnp.arange(n_extra, dtype=jnp.int32)
    src_p = jnp.concatenate([src, ar % N_NODES])
    dst_p = jnp.concatenate([dst, N_NODES + ar % (NPAD - N_NODES)])
    src2d = src_p.reshape(EPAD // CH, CH)
    dst2d = dst_p.reshape(EPAD // CH, CH)
    src2d_2 = jnp.stack([src2d, src2d + NPAD])

    x_p = jnp.zeros((NPAD, D_IN), _F32).at[:N_NODES].set(x)

    cnts = _count()(dst2d)
    sums1 = _agg()(x_p, src2d, dst2d)

    b1l2 = b1l.reshape(1, D_HID)
    b2l2 = b2l.reshape(1, D_HID)
    h = _tc1(sums1, cnts, x_p, W1l, b1l2, W1r)

    s2 = _agg2()(h.reshape(2 * NPAD, D_IN), src2d_2, dst2d)

    wc_p = jnp.zeros((D_HID, D_IN), _F32).at[:, :1].set(Wc)
    bc_p = jnp.zeros((1, D_IN), _F32).at[:, :1].set(bc.reshape(1, 1))
    out = _tc2(s2, cnts, h, W2l, b2l2, W2r, wc_p, bc_p)
    return out[:N_NODES, :1]
